# Initial kernel scaffold; baseline (speedup 1.0000x reference)
import jax
import jax.numpy as jnp
from jax.experimental import pallas as pl


def kernel(weights, s_offsets):
    raise NotImplementedError("write your pallas kernel here")


# speedup vs baseline: 5.9398x; 5.9398x over previous
import functools

import jax
import jax.numpy as jnp
from jax import lax
from jax.experimental import pallas as pl
from jax.experimental.pallas import tpu as pltpu
from jax.experimental.pallas import tpu_sc as plsc

_R = 65536
_D = 128
_NS = 64
_OUT = _D + _NS
_EPS = 1e-5
_NC = 2
_NSUB = 16
_NW = _NC * _NSUB
_RPW = _R // _NW
_G = 16
_NGRP = _RPW // _G
_HS = 66


def _sc_body(w_hbm, s_hbm, out_hbm, w_v, s_v, buf_v, hist_v, out_v):
    wid = lax.axis_index("s") * _NC + lax.axis_index("c")
    lane = lax.iota(jnp.int32, 16)
    b128 = lane * _D
    b129 = lane * (_D + 1)
    b192 = lane * _OUT
    b66 = lane * _HS
    zeros_i = jnp.zeros((16,), jnp.int32)
    zeros_f = jnp.zeros((16,), jnp.float32)
    ones_i = jnp.ones((16,), jnp.int32)

    def _zh(m, c):
        plsc.store_scatter(hist_v, [b66 + m], zeros_i)
        return c

    lax.fori_loop(0, _NS, _zh, 0)

    def _group(g, c):
        row0 = wid * _RPW + g * _G
        pltpu.sync_copy(w_hbm.at[pl.ds(row0 * _D, _G * _D)], w_v)
        pltpu.sync_copy(s_hbm.at[pl.ds(row0 * _D, _G * _D)], s_v)

        plsc.store_scatter(buf_v, [b129], zeros_f)

        def _pass_a(k, acc):
            wk = plsc.load_gather(w_v, [b128 + k])
            acc = acc + wk
            plsc.store_scatter(buf_v, [b129 + (k + 1)], acc)
            return acc

        s_total = lax.fori_loop(0, _D, _pass_a, zeros_f)
        padv = jnp.maximum(_EPS - s_total, 0.0) * (1.0 / _D)
        denom = s_total + padv * _D

        def _pass_b(k, prev_p):
            craw = plsc.load_gather(buf_v, [b129 + (k + 1)])
            kp1 = (k + 1).astype(jnp.float32)
            cdf = (craw + kp1 * padv) / denom
            plsc.store_scatter(buf_v, [b129 + (k + 1)], cdf)
            x = (cdf * 128.0 - 1.0) * 0.5
            xi = x.astype(jnp.int32)
            p = jnp.where(x > xi.astype(jnp.float32), xi + 1, xi)
            p = jnp.clip(p, 0, _NS)
            sk = plsc.load_gather(s_v, [b128 + k])
            plsc.store_scatter(out_v, [b192 + k + prev_p], sk)
            plsc.addupdate_scatter(hist_v, [b66 + p], ones_i)
            return p

        lax.fori_loop(0, _D, _pass_b, zeros_i)

        def _pass_c(j, accn):
            h = plsc.load_gather(hist_v, [b66 + j])
            plsc.store_scatter(hist_v, [b66 + j], zeros_i)
            n = accn + h
            nr = jnp.minimum(n + 1, _D - 1)
            cl = plsc.load_gather(buf_v, [b129 + n])
            cr = plsc.load_gather(buf_v, [b129 + nr])
            ol = plsc.load_gather(s_v, [b128 + n])
            orr = plsc.load_gather(s_v, [b128 + nr])
            u = (j.astype(jnp.float32) * 2.0 + 1.0) * (1.0 / 128.0)
            dd = cr - cl
            t = jnp.where(dd > 0, (u - cl) / dd, 0.0)
            t = jnp.clip(t, 0.0, 1.0)
            nv = ol + t * (orr - ol)
            plsc.store_scatter(out_v, [b192 + (j + 1) + n], nv)
            return n

        lax.fori_loop(0, _NS, _pass_c, zeros_i)

        pltpu.sync_copy(out_v, out_hbm.at[pl.ds(row0 * _OUT, _G * _OUT)])
        return c

    lax.fori_loop(0, _NGRP, _group, 0)


_sc_kernel = functools.partial(
    pl.kernel,
    out_type=jax.ShapeDtypeStruct((_R * _OUT,), jnp.float32),
    mesh=plsc.VectorSubcoreMesh(
        core_axis_name="c", subcore_axis_name="s",
        num_cores=_NC, num_subcores=_NSUB),
    compiler_params=pltpu.CompilerParams(needs_layout_passes=False),
    scratch_types=[
        pltpu.VMEM((_G * _D,), jnp.float32),
        pltpu.VMEM((_G * _D,), jnp.float32),
        pltpu.VMEM((_G * (_D + 1),), jnp.float32),
        pltpu.VMEM((_G * _HS,), jnp.int32),
        pltpu.VMEM((_G * _OUT,), jnp.float32),
    ],
)(_sc_body)


def kernel(weights, s_offsets):
    out = _sc_kernel(weights.reshape(-1), s_offsets.reshape(-1))
    return out.reshape(_R, _OUT)
